# Initial kernel scaffold; baseline (speedup 1.0000x reference)
import jax
import jax.numpy as jnp
from jax.experimental import pallas as pl


def kernel(x, edge_index, batch, W1, b1, W2, b2, W3, b3, Wfc, bfc):
    raise NotImplementedError("write your pallas kernel here")


# speedup vs baseline: 12.2245x; 12.2245x over previous
import functools

import jax
import jax.numpy as jnp
from jax import lax
from jax.experimental import pallas as pl
from jax.experimental.pallas import tpu as pltpu
from jax.experimental.pallas import tpu_sc as plsc

_N = 10000
_E = 320000
_DIN = 128
_H = 64
_G = 16

_NC, _NS = 2, 16
_CH = 128
_KCH = 80
_EPT = _CH * _KCH
_EPAD = _EPT * _NC * _NS
_NPAD = 10112
_STRIPE = _NPAD // _NS
_BLK = 1000


def _sc_mesh():
    return plsc.VectorSubcoreMesh(core_axis_name="c", subcore_axis_name="s",
                                  num_cores=_NC, num_subcores=_NS)


@functools.cache
def _deg_hist_kernel():
    return pl.kernel(
        _deg_hist_body,
        out_type=jax.ShapeDtypeStruct((_NC, _NPAD, 16), jnp.float32),
        mesh=_sc_mesh(),
        scratch_types=[
            pltpu.VMEM((_KCH, _CH), jnp.int32),
            pltpu.VMEM((_CH, 16), jnp.float32),
            pltpu.VMEM((_CH, 16), jnp.float32),
            pltpu.VMEM_SHARED((_NPAD, 16), jnp.float32),
        ],
        compiler_params=pltpu.CompilerParams(use_tc_tiling_on_sc=False),
    )


def _deg_hist_body(dstg, out, dstv, onev, zb, acc):
    c = lax.axis_index("c")
    s = lax.axis_index("s")
    wid = c * _NS + s

    def _fill(r, carry):
        zb[r, pl.ds(0, 16)] = jnp.zeros((16,), jnp.float32)
        onev[r, pl.ds(0, 16)] = jnp.ones((16,), jnp.float32)
        return carry

    lax.fori_loop(0, _CH, _fill, 0)
    row0 = s * _STRIPE
    for j in range(_STRIPE // _CH):
        pltpu.sync_copy(zb, acc.at[pl.ds(row0 + j * _CH, _CH)])
    rem = _STRIPE % _CH
    if rem:
        pltpu.sync_copy(zb.at[pl.ds(0, rem)],
                        acc.at[pl.ds(row0 + (_STRIPE // _CH) * _CH, rem)])
    pltpu.sync_copy(dstg.at[wid], dstv)
    plsc.subcore_barrier()

    def _step(k, carry):
        pltpu.sync_copy(onev, acc.at[dstv.at[k]], add=True)
        return carry

    lax.fori_loop(0, _KCH, _step, 0)
    plsc.subcore_barrier()
    pltpu.sync_copy(acc.at[pl.ds(row0, _STRIPE)], out.at[c, pl.ds(row0, _STRIPE)])


@functools.cache
def _edge_scatter_kernel():
    return pl.kernel(
        _edge_scatter_body,
        out_type=jax.ShapeDtypeStruct((_NC, _NPAD, _H), jnp.float32),
        mesh=_sc_mesh(),
        scratch_types=[
            pltpu.VMEM((_KCH, _CH), jnp.int32),
            pltpu.VMEM((_KCH, _CH), jnp.int32),
            pltpu.VMEM((_CH, _H), jnp.float32),
            pltpu.VMEM((_CH, _H), jnp.float32),
            pltpu.VMEM((_CH, _H), jnp.float32),
            pltpu.VMEM_SHARED((_NPAD, _H), jnp.float32),
            pltpu.SemaphoreType.DMA,
            pltpu.SemaphoreType.DMA,
        ],
        compiler_params=pltpu.CompilerParams(use_tc_tiling_on_sc=False),
    )


def _edge_scatter_body(hs, srcg, dstg, out, srcv, dstv, b0, b1, zb, acc, semA, semB):
    c = lax.axis_index("c")
    s = lax.axis_index("s")
    wid = c * _NS + s

    def _zrow(r, carry):
        for q in range(_H // 16):
            zb[r, pl.ds(q * 16, 16)] = jnp.zeros((16,), jnp.float32)
        return carry

    lax.fori_loop(0, _CH, _zrow, 0)
    row0 = s * _STRIPE
    for j in range(_STRIPE // _CH):
        pltpu.sync_copy(zb, acc.at[pl.ds(row0 + j * _CH, _CH)])
    rem = _STRIPE % _CH
    if rem:
        pltpu.sync_copy(zb.at[pl.ds(0, rem)],
                        acc.at[pl.ds(row0 + (_STRIPE // _CH) * _CH, rem)])
    pltpu.sync_copy(srcg.at[wid], srcv)
    pltpu.sync_copy(dstg.at[wid], dstv)
    plsc.subcore_barrier()

    pltpu.async_copy(hs.at[srcv.at[0]], b0, semA)

    def _step(i, carry):
        k = i * 2
        pltpu.make_async_copy(hs.at[srcv.at[k]], b0, semA).wait()
        pltpu.async_copy(hs.at[srcv.at[k + 1]], b1, semB)
        pltpu.sync_copy(b0, acc.at[dstv.at[k]], add=True)
        pltpu.make_async_copy(hs.at[srcv.at[k + 1]], b1, semB).wait()

        @pl.when(k + 2 < _KCH)
        def _():
            pltpu.async_copy(hs.at[srcv.at[k + 2]], b0, semA)

        pltpu.sync_copy(b1, acc.at[dstv.at[k + 1]], add=True)
        return carry

    lax.fori_loop(0, _KCH // 2, _step, 0)
    plsc.subcore_barrier()
    pltpu.sync_copy(acc.at[pl.ds(row0, _STRIPE)], out.at[c, pl.ds(row0, _STRIPE)])


def _dis_from(deg_ref):
    deg = 1.0 + deg_ref[0, :, 0] + deg_ref[1, :, 0]
    return lax.rsqrt(deg)


def _mm_scale_body(x_ref, w_ref, deg_ref, o_ref):
    dis = _dis_from(deg_ref)
    h = jnp.dot(x_ref[...], w_ref[...], preferred_element_type=jnp.float32)
    o_ref[...] = h * dis[:, None]


def _combine_mm_body(acc_ref, hs_ref, deg_ref, b_ref, w_ref, o_ref):
    dis = _dis_from(deg_ref)
    ssum = acc_ref[0] + acc_ref[1] + hs_ref[...]
    h = jnp.maximum(ssum * dis[:, None] + b_ref[...], 0.0)
    o_ref[...] = jnp.dot(h, w_ref[...], preferred_element_type=jnp.float32) * dis[:, None]


def _final_body(acc_ref, hs_ref, deg_ref, b_ref, bat_ref, wfc_ref, bfc_ref,
                o_ref, sums_ref, cnt_ref):
    i = pl.program_id(0)
    dis = _dis_from(deg_ref)
    ssum = acc_ref[0] + acc_ref[1] + hs_ref[...]
    h3 = jnp.maximum(ssum * dis[:, None] + b_ref[...], 0.0)
    bat = bat_ref[0, 0, :]
    gi = lax.broadcasted_iota(jnp.int32, (_G, _BLK), 0)
    oh = (gi == bat[None, :]).astype(jnp.float32)
    part = jnp.dot(oh, h3, preferred_element_type=jnp.float32)
    pcnt = jnp.sum(oh, axis=1)

    @pl.when(i == 0)
    def _():
        sums_ref[...] = part
        cnt_ref[...] = pcnt[None, :]

    @pl.when(i > 0)
    def _():
        sums_ref[...] += part
        cnt_ref[...] += pcnt[None, :]

    @pl.when(i == _N // _BLK - 1)
    def _():
        cnt = jnp.maximum(cnt_ref[0, :], 1.0)
        mean = sums_ref[...] / cnt[:, None]
        o_ref[...] = jnp.dot(mean, wfc_ref[...],
                             preferred_element_type=jnp.float32) + bfc_ref[...]


_GRID = (_N // _BLK,)
_DEG_SPEC = pl.BlockSpec((2, _BLK, 16), lambda i: (0, i, 0))
_ACC_SPEC = pl.BlockSpec((2, _BLK, _H), lambda i: (0, i, 0))
_ROW_SPEC = pl.BlockSpec((_BLK, _H), lambda i: (i, 0))


def _mm_scale(x, w, degp):
    return pl.pallas_call(
        _mm_scale_body, grid=_GRID,
        in_specs=[pl.BlockSpec((_BLK, _DIN), lambda i: (i, 0)),
                  pl.BlockSpec((_DIN, _H), lambda i: (0, 0)),
                  _DEG_SPEC],
        out_specs=_ROW_SPEC,
        out_shape=jax.ShapeDtypeStruct((_N, _H), jnp.float32),
    )(x, w, degp)


def _combine_mm(acc, hs, degp, b, w):
    return pl.pallas_call(
        _combine_mm_body, grid=_GRID,
        in_specs=[_ACC_SPEC, _ROW_SPEC, _DEG_SPEC,
                  pl.BlockSpec((1, _H), lambda i: (0, 0)),
                  pl.BlockSpec((_H, _H), lambda i: (0, 0))],
        out_specs=_ROW_SPEC,
        out_shape=jax.ShapeDtypeStruct((_N, _H), jnp.float32),
    )(acc, hs, degp, b, w)


def _final(acc, hs, degp, b, bat, wfc, bfc):
    return pl.pallas_call(
        _final_body, grid=_GRID,
        in_specs=[_ACC_SPEC, _ROW_SPEC, _DEG_SPEC,
                  pl.BlockSpec((1, _H), lambda i: (0, 0)),
                  pl.BlockSpec((1, 1, _BLK), lambda i: (i, 0, 0)),
                  pl.BlockSpec((_H, 2), lambda i: (0, 0)),
                  pl.BlockSpec((1, 2), lambda i: (0, 0))],
        out_specs=pl.BlockSpec((_G, 2), lambda i: (0, 0)),
        out_shape=jax.ShapeDtypeStruct((_G, 2), jnp.float32),
        scratch_shapes=[pltpu.VMEM((_G, _H), jnp.float32),
                        pltpu.VMEM((1, _G), jnp.float32)],
    )(acc, hs, degp, b, bat, wfc, bfc)


def kernel(x, edge_index, batch, W1, b1, W2, b2, W3, b3, Wfc, bfc):
    src = edge_index[0].astype(jnp.int32)
    dst = edge_index[1].astype(jnp.int32)
    pad = _EPAD - _E
    srcp = jnp.concatenate([src, jnp.zeros((pad,), jnp.int32)])
    dstp = jnp.concatenate([dst, jnp.full((pad,), _N, jnp.int32)])
    srcg = srcp.reshape(_NC * _NS, _KCH, _CH)
    dstg = dstp.reshape(_NC * _NS, _KCH, _CH)

    degp = _deg_hist_kernel()(dstg)
    hs1 = _mm_scale(x, W1, degp)
    acc1 = _edge_scatter_kernel()(hs1, srcg, dstg)
    hs2 = _combine_mm(acc1, hs1, degp, b1.reshape(1, _H), W2)
    acc2 = _edge_scatter_kernel()(hs2, srcg, dstg)
    hs3 = _combine_mm(acc2, hs2, degp, b2.reshape(1, _H), W3)
    acc3 = _edge_scatter_kernel()(hs3, srcg, dstg)
    batr = batch.astype(jnp.int32).reshape(_N // _BLK, 1, _BLK)
    return _final(acc3, hs3, degp, b3.reshape(1, _H), batr,
                  Wfc, bfc.reshape(1, 2))


# speedup vs baseline: 31.0268x; 2.5381x over previous
import functools

import jax
import jax.numpy as jnp
from jax import lax
from jax.experimental import pallas as pl
from jax.experimental.pallas import tpu as pltpu
from jax.experimental.pallas import tpu_sc as plsc

_N = 10000
_E = 320000
_DIN = 128
_H = 64
_G = 16

_NC, _NS = 2, 16
_CH = 128
_KCH = 80
_EPT = _CH * _KCH
_EPAD = _EPT * _NC * _NS
_NPAD = 10112
_STRIPE = _NPAD // _NS
_BLK = 1000


def _sc_mesh():
    return plsc.VectorSubcoreMesh(core_axis_name="c", subcore_axis_name="s",
                                  num_cores=_NC, num_subcores=_NS)


@functools.cache
def _deg_hist_kernel():
    return pl.kernel(
        _deg_hist_body,
        out_type=jax.ShapeDtypeStruct((_NC, _NPAD, 16), jnp.float32),
        mesh=_sc_mesh(),
        scratch_types=[
            pltpu.VMEM((_KCH, _CH), jnp.int32),
            pltpu.VMEM((_CH, 16), jnp.float32),
            pltpu.VMEM((_CH, 16), jnp.float32),
            pltpu.VMEM_SHARED((_NPAD, 16), jnp.float32),
        ],
        compiler_params=pltpu.CompilerParams(use_tc_tiling_on_sc=False),
    )


def _deg_hist_body(dstg, out, dstv, onev, zb, acc):
    c = lax.axis_index("c")
    s = lax.axis_index("s")
    wid = c * _NS + s

    def _fill(r, carry):
        zb[r, pl.ds(0, 16)] = jnp.zeros((16,), jnp.float32)
        onev[r, pl.ds(0, 16)] = jnp.ones((16,), jnp.float32)
        return carry

    lax.fori_loop(0, _CH, _fill, 0)
    row0 = s * _STRIPE
    for j in range(_STRIPE // _CH):
        pltpu.sync_copy(zb, acc.at[pl.ds(row0 + j * _CH, _CH)])
    rem = _STRIPE % _CH
    if rem:
        pltpu.sync_copy(zb.at[pl.ds(0, rem)],
                        acc.at[pl.ds(row0 + (_STRIPE // _CH) * _CH, rem)])
    pltpu.sync_copy(dstg.at[wid], dstv)
    plsc.subcore_barrier()

    def _step(k, carry):
        pltpu.sync_copy(onev, acc.at[dstv.at[k]], add=True)
        return carry

    lax.fori_loop(0, _KCH, _step, 0)
    plsc.subcore_barrier()
    pltpu.sync_copy(acc.at[pl.ds(row0, _STRIPE)], out.at[c, pl.ds(row0, _STRIPE)])


@functools.cache
def _edge_scatter_kernel():
    return pl.kernel(
        _edge_scatter_body,
        out_type=jax.ShapeDtypeStruct((_NC, _NPAD, _H), jnp.float32),
        mesh=_sc_mesh(),
        scratch_types=[
            pltpu.VMEM((_KCH, _CH), jnp.int32),
            pltpu.VMEM((_KCH, _CH), jnp.int32),
            pltpu.VMEM((_CH, _H), jnp.float32),
            pltpu.VMEM((_CH, _H), jnp.float32),
            pltpu.VMEM((_CH, _H), jnp.float32),
            pltpu.VMEM_SHARED((_NPAD, _H), jnp.float32),
            pltpu.VMEM_SHARED((_NPAD, _H), jnp.float32),
            pltpu.SemaphoreType.DMA,
            pltpu.SemaphoreType.DMA,
        ],
        compiler_params=pltpu.CompilerParams(use_tc_tiling_on_sc=False),
    )


def _edge_scatter_body(hs, srcg, dstg, out, srcv, dstv, b0, b1, zb, acc, hs_sh,
                       semA, semB):
    c = lax.axis_index("c")
    s = lax.axis_index("s")
    wid = c * _NS + s

    def _zrow(r, carry):
        for q in range(_H // 16):
            zb[r, pl.ds(q * 16, 16)] = jnp.zeros((16,), jnp.float32)
        return carry

    lax.fori_loop(0, _CH, _zrow, 0)
    row0 = s * _STRIPE
    for j in range(_STRIPE // _CH):
        pltpu.sync_copy(zb, acc.at[pl.ds(row0 + j * _CH, _CH)])
    rem = _STRIPE % _CH
    if rem:
        pltpu.sync_copy(zb.at[pl.ds(0, rem)],
                        acc.at[pl.ds(row0 + (_STRIPE // _CH) * _CH, rem)])
    pltpu.sync_copy(srcg.at[wid], srcv)
    pltpu.sync_copy(dstg.at[wid], dstv)
    pltpu.sync_copy(hs.at[pl.ds(row0, _STRIPE)], hs_sh.at[pl.ds(row0, _STRIPE)])
    plsc.subcore_barrier()

    pltpu.async_copy(hs_sh.at[srcv.at[0]], b0, semA)

    def _step(i, carry):
        k = i * 2
        pltpu.make_async_copy(hs_sh.at[srcv.at[k]], b0, semA).wait()
        pltpu.async_copy(hs_sh.at[srcv.at[k + 1]], b1, semB)
        pltpu.sync_copy(b0, acc.at[dstv.at[k]], add=True)
        pltpu.make_async_copy(hs_sh.at[srcv.at[k + 1]], b1, semB).wait()

        @pl.when(k + 2 < _KCH)
        def _():
            pltpu.async_copy(hs_sh.at[srcv.at[k + 2]], b0, semA)

        pltpu.sync_copy(b1, acc.at[dstv.at[k + 1]], add=True)
        return carry

    lax.fori_loop(0, _KCH // 2, _step, 0)
    plsc.subcore_barrier()
    pltpu.sync_copy(acc.at[pl.ds(row0, _STRIPE)], out.at[c, pl.ds(row0, _STRIPE)])


def _dis_from(deg_ref):
    deg = 1.0 + deg_ref[0, :, 0] + deg_ref[1, :, 0]
    return lax.rsqrt(deg)


def _mm_scale_body(x_ref, w_ref, deg_ref, o_ref):
    dis = _dis_from(deg_ref)
    h = jnp.dot(x_ref[...], w_ref[...], preferred_element_type=jnp.float32)
    o_ref[...] = h * dis[:, None]


def _combine_mm_body(acc_ref, hs_ref, deg_ref, b_ref, w_ref, o_ref):
    dis = _dis_from(deg_ref)
    ssum = acc_ref[0] + acc_ref[1] + hs_ref[...]
    h = jnp.maximum(ssum * dis[:, None] + b_ref[...], 0.0)
    o_ref[...] = jnp.dot(h, w_ref[...], preferred_element_type=jnp.float32) * dis[:, None]


def _final_body(acc_ref, hs_ref, deg_ref, b_ref, bat_ref, wfc_ref, bfc_ref,
                o_ref, sums_ref, cnt_ref):
    i = pl.program_id(0)
    dis = _dis_from(deg_ref)
    ssum = acc_ref[0] + acc_ref[1] + hs_ref[...]
    h3 = jnp.maximum(ssum * dis[:, None] + b_ref[...], 0.0)
    bat = bat_ref[0, 0, :]
    gi = lax.broadcasted_iota(jnp.int32, (_G, _BLK), 0)
    oh = (gi == bat[None, :]).astype(jnp.float32)
    part = jnp.dot(oh, h3, preferred_element_type=jnp.float32)
    pcnt = jnp.sum(oh, axis=1)

    @pl.when(i == 0)
    def _():
        sums_ref[...] = part
        cnt_ref[...] = pcnt[None, :]

    @pl.when(i > 0)
    def _():
        sums_ref[...] += part
        cnt_ref[...] += pcnt[None, :]

    @pl.when(i == _N // _BLK - 1)
    def _():
        cnt = jnp.maximum(cnt_ref[0, :], 1.0)
        mean = sums_ref[...] / cnt[:, None]
        o_ref[...] = jnp.dot(mean, wfc_ref[...],
                             preferred_element_type=jnp.float32) + bfc_ref[...]


_GRID = (_N // _BLK,)
_DEG_SPEC = pl.BlockSpec((2, _BLK, 16), lambda i: (0, i, 0))
_ACC_SPEC = pl.BlockSpec((2, _BLK, _H), lambda i: (0, i, 0))
_ROW_SPEC = pl.BlockSpec((_BLK, _H), lambda i: (i, 0))


def _mm_scale(x, w, degp):
    return pl.pallas_call(
        _mm_scale_body, grid=_GRID,
        in_specs=[pl.BlockSpec((_BLK, _DIN), lambda i: (i, 0)),
                  pl.BlockSpec((_DIN, _H), lambda i: (0, 0)),
                  _DEG_SPEC],
        out_specs=_ROW_SPEC,
        out_shape=jax.ShapeDtypeStruct((_NPAD, _H), jnp.float32),
    )(x, w, degp)


def _combine_mm(acc, hs, degp, b, w):
    return pl.pallas_call(
        _combine_mm_body, grid=_GRID,
        in_specs=[_ACC_SPEC, _ROW_SPEC, _DEG_SPEC,
                  pl.BlockSpec((1, _H), lambda i: (0, 0)),
                  pl.BlockSpec((_H, _H), lambda i: (0, 0))],
        out_specs=_ROW_SPEC,
        out_shape=jax.ShapeDtypeStruct((_NPAD, _H), jnp.float32),
    )(acc, hs, degp, b, w)


def _final(acc, hs, degp, b, bat, wfc, bfc):
    return pl.pallas_call(
        _final_body, grid=_GRID,
        in_specs=[_ACC_SPEC, _ROW_SPEC, _DEG_SPEC,
                  pl.BlockSpec((1, _H), lambda i: (0, 0)),
                  pl.BlockSpec((1, 1, _BLK), lambda i: (i, 0, 0)),
                  pl.BlockSpec((_H, 2), lambda i: (0, 0)),
                  pl.BlockSpec((1, 2), lambda i: (0, 0))],
        out_specs=pl.BlockSpec((_G, 2), lambda i: (0, 0)),
        out_shape=jax.ShapeDtypeStruct((_G, 2), jnp.float32),
        scratch_shapes=[pltpu.VMEM((_G, _H), jnp.float32),
                        pltpu.VMEM((1, _G), jnp.float32)],
    )(acc, hs, degp, b, bat, wfc, bfc)


def kernel(x, edge_index, batch, W1, b1, W2, b2, W3, b3, Wfc, bfc):
    src = edge_index[0].astype(jnp.int32)
    dst = edge_index[1].astype(jnp.int32)
    pad = _EPAD - _E
    srcp = jnp.concatenate([src, jnp.zeros((pad,), jnp.int32)])
    dstp = jnp.concatenate([dst, jnp.full((pad,), _N, jnp.int32)])
    srcg = srcp.reshape(_NC * _NS, _KCH, _CH)
    dstg = dstp.reshape(_NC * _NS, _KCH, _CH)

    degp = _deg_hist_kernel()(dstg)
    hs1 = _mm_scale(x, W1, degp)
    acc1 = _edge_scatter_kernel()(hs1, srcg, dstg)
    hs2 = _combine_mm(acc1, hs1, degp, b1.reshape(1, _H), W2)
    acc2 = _edge_scatter_kernel()(hs2, srcg, dstg)
    hs3 = _combine_mm(acc2, hs2, degp, b2.reshape(1, _H), W3)
    acc3 = _edge_scatter_kernel()(hs3, srcg, dstg)
    batr = batch.astype(jnp.int32).reshape(_N // _BLK, 1, _BLK)
    return _final(acc3, hs3, degp, b3.reshape(1, _H), batr,
                  Wfc, bfc.reshape(1, 2))


# speedup vs baseline: 32.4065x; 1.0445x over previous
import functools

import jax
import jax.numpy as jnp
from jax import lax
from jax.experimental import pallas as pl
from jax.experimental.pallas import tpu as pltpu
from jax.experimental.pallas import tpu_sc as plsc

_N = 10000
_E = 320000
_DIN = 128
_H = 64
_G = 16

_NC, _NS = 2, 16
_CH = 125
_KCH = 80
_NPAD = 10112
_STRIPE = _NPAD // _NS
_BLK = 2000


def _sc_mesh():
    return plsc.VectorSubcoreMesh(core_axis_name="c", subcore_axis_name="s",
                                  num_cores=_NC, num_subcores=_NS)


@functools.cache
def _deg_hist_kernel():
    return pl.kernel(
        _deg_hist_body,
        out_type=jax.ShapeDtypeStruct((_NC, _NPAD, 16), jnp.float32),
        mesh=_sc_mesh(),
        scratch_types=[
            pltpu.VMEM((_KCH, _CH), jnp.int32),
            pltpu.VMEM((_CH, 16), jnp.float32),
            pltpu.VMEM((_CH, 16), jnp.float32),
            pltpu.VMEM_SHARED((_NPAD, 16), jnp.float32),
        ],
        compiler_params=pltpu.CompilerParams(use_tc_tiling_on_sc=False),
    )


def _deg_hist_body(dstg, out, dstv, onev, zb, acc):
    c = lax.axis_index("c")
    s = lax.axis_index("s")
    wid = c * _NS + s

    def _fill(r, carry):
        zb[r, pl.ds(0, 16)] = jnp.zeros((16,), jnp.float32)
        onev[r, pl.ds(0, 16)] = jnp.ones((16,), jnp.float32)
        return carry

    lax.fori_loop(0, _CH, _fill, 0)
    row0 = s * _STRIPE
    for j in range(_STRIPE // _CH):
        pltpu.sync_copy(zb, acc.at[pl.ds(row0 + j * _CH, _CH)])
    rem = _STRIPE % _CH
    if rem:
        pltpu.sync_copy(zb.at[pl.ds(0, rem)],
                        acc.at[pl.ds(row0 + (_STRIPE // _CH) * _CH, rem)])
    pltpu.sync_copy(dstg.at[wid], dstv)
    plsc.subcore_barrier()

    def _step(k, carry):
        pltpu.sync_copy(onev, acc.at[dstv.at[k]], add=True)
        return carry

    lax.fori_loop(0, _KCH, _step, 0)
    plsc.subcore_barrier()
    pltpu.sync_copy(acc.at[pl.ds(row0, _STRIPE)], out.at[c, pl.ds(row0, _STRIPE)])


@functools.cache
def _edge_scatter_kernel():
    return pl.kernel(
        _edge_scatter_body,
        out_type=jax.ShapeDtypeStruct((_NC, _NPAD, _H), jnp.float32),
        mesh=_sc_mesh(),
        scratch_types=[
            pltpu.VMEM((_KCH, _CH), jnp.int32),
            pltpu.VMEM((_KCH, _CH), jnp.int32),
            pltpu.VMEM((_CH, _H), jnp.float32),
            pltpu.VMEM((_CH, _H), jnp.float32),
            pltpu.VMEM((_CH, _H), jnp.float32),
            pltpu.VMEM_SHARED((_NPAD, _H), jnp.float32),
            pltpu.VMEM_SHARED((_NPAD, _H), jnp.float32),
            pltpu.SemaphoreType.DMA,
            pltpu.SemaphoreType.DMA,
        ],
        compiler_params=pltpu.CompilerParams(use_tc_tiling_on_sc=False),
    )


def _edge_scatter_body(hs, srcg, dstg, out, srcv, dstv, b0, b1, zb, acc, hs_sh,
                       semA, semB):
    c = lax.axis_index("c")
    s = lax.axis_index("s")
    wid = c * _NS + s

    def _zrow(r, carry):
        for q in range(_H // 16):
            zb[r, pl.ds(q * 16, 16)] = jnp.zeros((16,), jnp.float32)
        return carry

    lax.fori_loop(0, _CH, _zrow, 0)
    row0 = s * _STRIPE
    for j in range(_STRIPE // _CH):
        pltpu.sync_copy(zb, acc.at[pl.ds(row0 + j * _CH, _CH)])
    rem = _STRIPE % _CH
    if rem:
        pltpu.sync_copy(zb.at[pl.ds(0, rem)],
                        acc.at[pl.ds(row0 + (_STRIPE // _CH) * _CH, rem)])
    pltpu.sync_copy(srcg.at[wid], srcv)
    pltpu.sync_copy(dstg.at[wid], dstv)
    pltpu.sync_copy(hs.at[pl.ds(row0, _STRIPE)], hs_sh.at[pl.ds(row0, _STRIPE)])
    plsc.subcore_barrier()

    pltpu.async_copy(hs_sh.at[srcv.at[0]], b0, semA)

    def _step(i, carry):
        k = i * 2
        pltpu.make_async_copy(hs_sh.at[srcv.at[k]], b0, semA).wait()
        pltpu.async_copy(hs_sh.at[srcv.at[k + 1]], b1, semB)
        pltpu.sync_copy(b0, acc.at[dstv.at[k]], add=True)
        pltpu.make_async_copy(hs_sh.at[srcv.at[k + 1]], b1, semB).wait()

        @pl.when(k + 2 < _KCH)
        def _():
            pltpu.async_copy(hs_sh.at[srcv.at[k + 2]], b0, semA)

        pltpu.sync_copy(b1, acc.at[dstv.at[k + 1]], add=True)
        return carry

    lax.fori_loop(0, _KCH // 2, _step, 0)
    plsc.subcore_barrier()
    pltpu.sync_copy(acc.at[pl.ds(row0, _STRIPE)], out.at[c, pl.ds(row0, _STRIPE)])


def _dis_from(deg_ref):
    deg = 1.0 + deg_ref[0, :, 0] + deg_ref[1, :, 0]
    return lax.rsqrt(deg)


def _mm_scale_body(x_ref, w_ref, deg_ref, o_ref):
    dis = _dis_from(deg_ref)
    h = jnp.dot(x_ref[...], w_ref[...], preferred_element_type=jnp.float32)
    o_ref[...] = h * dis[:, None]


def _combine_mm_body(acc_ref, hs_ref, deg_ref, b_ref, w_ref, o_ref):
    dis = _dis_from(deg_ref)
    ssum = acc_ref[0] + acc_ref[1] + hs_ref[...]
    h = jnp.maximum(ssum * dis[:, None] + b_ref[...], 0.0)
    o_ref[...] = jnp.dot(h, w_ref[...], preferred_element_type=jnp.float32) * dis[:, None]


def _final_body(acc_ref, hs_ref, deg_ref, b_ref, bat_ref, wfc_ref, bfc_ref,
                o_ref, sums_ref, cnt_ref):
    i = pl.program_id(0)
    dis = _dis_from(deg_ref)
    ssum = acc_ref[0] + acc_ref[1] + hs_ref[...]
    h3 = jnp.maximum(ssum * dis[:, None] + b_ref[...], 0.0)
    bat = bat_ref[0, 0, :]
    gi = lax.broadcasted_iota(jnp.int32, (_G, _BLK), 0)
    oh = (gi == bat[None, :]).astype(jnp.float32)
    part = jnp.dot(oh, h3, preferred_element_type=jnp.float32)
    pcnt = jnp.sum(oh, axis=1)

    @pl.when(i == 0)
    def _():
        sums_ref[...] = part
        cnt_ref[...] = pcnt[None, :]

    @pl.when(i > 0)
    def _():
        sums_ref[...] += part
        cnt_ref[...] += pcnt[None, :]

    @pl.when(i == _N // _BLK - 1)
    def _():
        cnt = jnp.maximum(cnt_ref[0, :], 1.0)
        mean = sums_ref[...] / cnt[:, None]
        o_ref[...] = jnp.dot(mean, wfc_ref[...],
                             preferred_element_type=jnp.float32) + bfc_ref[...]


_GRID = (_N // _BLK,)
_DEG_SPEC = pl.BlockSpec((2, _BLK, 16), lambda i: (0, i, 0))
_ACC_SPEC = pl.BlockSpec((2, _BLK, _H), lambda i: (0, i, 0))
_ROW_SPEC = pl.BlockSpec((_BLK, _H), lambda i: (i, 0))


def _mm_scale(x, w, degp):
    return pl.pallas_call(
        _mm_scale_body, grid=_GRID,
        in_specs=[pl.BlockSpec((_BLK, _DIN), lambda i: (i, 0)),
                  pl.BlockSpec((_DIN, _H), lambda i: (0, 0)),
                  _DEG_SPEC],
        out_specs=_ROW_SPEC,
        out_shape=jax.ShapeDtypeStruct((_NPAD, _H), jnp.float32),
    )(x, w, degp)


def _combine_mm(acc, hs, degp, b, w):
    return pl.pallas_call(
        _combine_mm_body, grid=_GRID,
        in_specs=[_ACC_SPEC, _ROW_SPEC, _DEG_SPEC,
                  pl.BlockSpec((1, _H), lambda i: (0, 0)),
                  pl.BlockSpec((_H, _H), lambda i: (0, 0))],
        out_specs=_ROW_SPEC,
        out_shape=jax.ShapeDtypeStruct((_NPAD, _H), jnp.float32),
    )(acc, hs, degp, b, w)


def _final(acc, hs, degp, b, bat, wfc, bfc):
    return pl.pallas_call(
        _final_body, grid=_GRID,
        in_specs=[_ACC_SPEC, _ROW_SPEC, _DEG_SPEC,
                  pl.BlockSpec((1, _H), lambda i: (0, 0)),
                  pl.BlockSpec((1, 1, _BLK), lambda i: (i, 0, 0)),
                  pl.BlockSpec((_H, 2), lambda i: (0, 0)),
                  pl.BlockSpec((1, 2), lambda i: (0, 0))],
        out_specs=pl.BlockSpec((_G, 2), lambda i: (0, 0)),
        out_shape=jax.ShapeDtypeStruct((_G, 2), jnp.float32),
        scratch_shapes=[pltpu.VMEM((_G, _H), jnp.float32),
                        pltpu.VMEM((1, _G), jnp.float32)],
    )(acc, hs, degp, b, bat, wfc, bfc)


def kernel(x, edge_index, batch, W1, b1, W2, b2, W3, b3, Wfc, bfc):
    srcg = edge_index[0].astype(jnp.int32).reshape(_NC * _NS, _KCH, _CH)
    dstg = edge_index[1].astype(jnp.int32).reshape(_NC * _NS, _KCH, _CH)

    degp = _deg_hist_kernel()(dstg)
    hs1 = _mm_scale(x, W1, degp)
    acc1 = _edge_scatter_kernel()(hs1, srcg, dstg)
    hs2 = _combine_mm(acc1, hs1, degp, b1.reshape(1, _H), W2)
    acc2 = _edge_scatter_kernel()(hs2, srcg, dstg)
    hs3 = _combine_mm(acc2, hs2, degp, b2.reshape(1, _H), W3)
    acc3 = _edge_scatter_kernel()(hs3, srcg, dstg)
    batr = batch.astype(jnp.int32).reshape(_N // _BLK, 1, _BLK)
    return _final(acc3, hs3, degp, b3.reshape(1, _H), batr,
                  Wfc, bfc.reshape(1, 2))


# speedup vs baseline: 36.5524x; 1.1279x over previous
import functools

import jax
import jax.numpy as jnp
from jax import lax
from jax.experimental import pallas as pl
from jax.experimental.pallas import tpu as pltpu
from jax.experimental.pallas import tpu_sc as plsc

_N = 10000
_E = 320000
_DIN = 128
_H = 64
_G = 16

_NC, _NS = 2, 16
_CH = 125
_KCH = 80
_NPAD = 10112
_STRIPE = _NPAD // _NS
_NP2 = _NPAD // 2
_NGRID = 8
_PBLK = _NP2 // _NGRID


def _sc_mesh():
    return plsc.VectorSubcoreMesh(core_axis_name="c", subcore_axis_name="s",
                                  num_cores=_NC, num_subcores=_NS)


@functools.cache
def _deg_hist_kernel():
    return pl.kernel(
        _deg_hist_body,
        out_type=jax.ShapeDtypeStruct((_NC, _NPAD, 16), jnp.float32),
        mesh=_sc_mesh(),
        scratch_types=[
            pltpu.VMEM((_KCH, _CH), jnp.int32),
            pltpu.VMEM((_CH, 16), jnp.float32),
            pltpu.VMEM((_CH, 16), jnp.float32),
            pltpu.VMEM_SHARED((_NPAD, 16), jnp.float32),
        ],
        compiler_params=pltpu.CompilerParams(use_tc_tiling_on_sc=False),
    )


def _deg_hist_body(dstg, out, dstv, onev, zb, acc):
    c = lax.axis_index("c")
    s = lax.axis_index("s")
    wid = c * _NS + s

    def _fill(r, carry):
        zb[r, pl.ds(0, 16)] = jnp.zeros((16,), jnp.float32)
        onev[r, pl.ds(0, 16)] = jnp.ones((16,), jnp.float32)
        return carry

    lax.fori_loop(0, _CH, _fill, 0)
    row0 = s * _STRIPE
    for j in range(_STRIPE // _CH):
        pltpu.sync_copy(zb, acc.at[pl.ds(row0 + j * _CH, _CH)])
    rem = _STRIPE % _CH
    if rem:
        pltpu.sync_copy(zb.at[pl.ds(0, rem)],
                        acc.at[pl.ds(row0 + (_STRIPE // _CH) * _CH, rem)])
    pltpu.sync_copy(dstg.at[wid], dstv)
    plsc.subcore_barrier()

    def _step(k, carry):
        pltpu.sync_copy(onev, acc.at[dstv.at[k]], add=True)
        return carry

    lax.fori_loop(0, _KCH, _step, 0)
    plsc.subcore_barrier()
    pltpu.sync_copy(acc.at[pl.ds(row0, _STRIPE)], out.at[c, pl.ds(row0, _STRIPE)])


@functools.cache
def _edge_scatter_kernel():
    return pl.kernel(
        _edge_scatter_body,
        out_type=jax.ShapeDtypeStruct((_NC, _NPAD, _H), jnp.float32),
        mesh=_sc_mesh(),
        scratch_types=[
            pltpu.VMEM((_KCH, _CH), jnp.int32),
            pltpu.VMEM((_KCH, _CH), jnp.int32),
            pltpu.VMEM((_CH, _H), jnp.float32),
            pltpu.VMEM((_CH, _H), jnp.float32),
            pltpu.VMEM((_CH, _H), jnp.float32),
            pltpu.VMEM_SHARED((_NPAD, _H), jnp.float32),
            pltpu.VMEM_SHARED((_NPAD, _H), jnp.float32),
            pltpu.SemaphoreType.DMA,
            pltpu.SemaphoreType.DMA,
        ],
        compiler_params=pltpu.CompilerParams(use_tc_tiling_on_sc=False),
    )


def _edge_scatter_body(hs, srcg, dstg, out, srcv, dstv, b0, b1, zb, acc, hs_sh,
                       semA, semB):
    c = lax.axis_index("c")
    s = lax.axis_index("s")
    wid = c * _NS + s

    def _zrow(r, carry):
        for q in range(_H // 16):
            zb[r, pl.ds(q * 16, 16)] = jnp.zeros((16,), jnp.float32)
        return carry

    lax.fori_loop(0, _CH, _zrow, 0)
    row0 = s * _STRIPE
    for j in range(_STRIPE // _CH):
        pltpu.sync_copy(zb, acc.at[pl.ds(row0 + j * _CH, _CH)])
    rem = _STRIPE % _CH
    if rem:
        pltpu.sync_copy(zb.at[pl.ds(0, rem)],
                        acc.at[pl.ds(row0 + (_STRIPE // _CH) * _CH, rem)])
    pltpu.sync_copy(srcg.at[wid], srcv)
    pltpu.sync_copy(dstg.at[wid], dstv)
    pltpu.sync_copy(hs.at[pl.ds(row0, _STRIPE)], hs_sh.at[pl.ds(row0, _STRIPE)])
    plsc.subcore_barrier()

    pltpu.async_copy(hs_sh.at[srcv.at[0]], b0, semA)

    def _step(i, carry):
        k = i * 2
        pltpu.make_async_copy(hs_sh.at[srcv.at[k]], b0, semA).wait()
        pltpu.async_copy(hs_sh.at[srcv.at[k + 1]], b1, semB)
        pltpu.sync_copy(b0, acc.at[dstv.at[k]], add=True)
        pltpu.make_async_copy(hs_sh.at[srcv.at[k + 1]], b1, semB).wait()

        @pl.when(k + 2 < _KCH)
        def _():
            pltpu.async_copy(hs_sh.at[srcv.at[k + 2]], b0, semA)

        pltpu.sync_copy(b1, acc.at[dstv.at[k + 1]], add=True)
        return carry

    lax.fori_loop(0, _KCH // 2, _step, 0)
    plsc.subcore_barrier()
    pltpu.sync_copy(acc.at[pl.ds(row0, _STRIPE)], out.at[c, pl.ds(row0, _STRIPE)])


def _dis_prep_body(deg_ref, o_ref):
    deg = 1.0 + deg_ref[0, :, 0] + deg_ref[1, :, 0]
    dis = lax.rsqrt(deg)
    lo = jnp.broadcast_to(dis[:_NP2, None], (_NP2, _H))
    hi = jnp.broadcast_to(dis[_NP2:, None], (_NP2, _H))
    o_ref[...] = jnp.concatenate([lo, hi], axis=1)


def _mm_scale_body(xlo_ref, xhi_ref, w_ref, disp_ref, o_ref):
    i = pl.program_id(0)
    w = w_ref[...]
    dis_p = disp_ref[...]
    hlo = jnp.dot(xlo_ref[...], w, preferred_element_type=jnp.float32)
    hhi = jnp.dot(xhi_ref[...], w, preferred_element_type=jnp.float32)
    node_hi = _NP2 + i * _PBLK + lax.broadcasted_iota(jnp.int32, (_PBLK, _H), 0)
    hhi = jnp.where(node_hi < _N, hhi, 0.0)
    o_ref[...] = jnp.concatenate([hlo, hhi], axis=1) * dis_p


def _combine_mm_body(acc_ref, hs_ref, disp_ref, b_ref, w_ref, o_ref):
    dis_p = disp_ref[...]
    ssum = acc_ref[0] + acc_ref[1] + hs_ref[...]
    h = jnp.maximum(ssum * dis_p + b_ref[...], 0.0)
    o_ref[...] = jnp.dot(h, w_ref[...], preferred_element_type=jnp.float32) * dis_p


def _final_body(acc_ref, hs_ref, disp_ref, b_ref, blo_ref, bhi_ref, wfc_ref,
                bfc_ref, o_ref, sums_ref, cnt_ref):
    i = pl.program_id(0)
    dis_p = disp_ref[...]
    ssum = acc_ref[0] + acc_ref[1] + hs_ref[...]
    h3 = jnp.maximum(ssum * dis_p + b_ref[...], 0.0)
    gi = lax.broadcasted_iota(jnp.int32, (_G, _PBLK), 0)
    ohe = (gi == blo_ref[0, 0, :][None, :]).astype(jnp.float32)
    oho = (gi == bhi_ref[0, 0, :][None, :]).astype(jnp.float32)
    part = (jnp.dot(ohe, h3[:, :_H], preferred_element_type=jnp.float32)
            + jnp.dot(oho, h3[:, _H:], preferred_element_type=jnp.float32))
    pcnt = jnp.sum(ohe, axis=1) + jnp.sum(oho, axis=1)

    @pl.when(i == 0)
    def _():
        sums_ref[...] = part
        cnt_ref[...] = pcnt[None, :]

    @pl.when(i > 0)
    def _():
        sums_ref[...] += part
        cnt_ref[...] += pcnt[None, :]

    @pl.when(i == _NGRID - 1)
    def _():
        cnt = jnp.maximum(cnt_ref[0, :], 1.0)
        mean = sums_ref[...] / cnt[:, None]
        o_ref[...] = jnp.dot(mean, wfc_ref[...],
                             preferred_element_type=jnp.float32) + bfc_ref[...]


_GRID = (_NGRID,)
_ACC_SPEC = pl.BlockSpec((2, _PBLK, 128), lambda i: (0, i, 0))
_ROW_SPEC = pl.BlockSpec((_PBLK, 128), lambda i: (i, 0))


def _dis_prep(degp):
    return pl.pallas_call(
        _dis_prep_body, grid=(1,),
        in_specs=[pl.BlockSpec((2, _NPAD, 16), lambda i: (0, 0, 0))],
        out_specs=pl.BlockSpec((_NP2, 128), lambda i: (0, 0)),
        out_shape=jax.ShapeDtypeStruct((_NP2, 128), jnp.float32),
    )(degp)


def _mm_scale(x, w, disp):
    return pl.pallas_call(
        _mm_scale_body, grid=_GRID,
        in_specs=[pl.BlockSpec((_PBLK, _DIN), lambda i: (i, 0)),
                  pl.BlockSpec((_PBLK, _DIN), lambda i: (i + _NGRID, 0)),
                  pl.BlockSpec((_DIN, _H), lambda i: (0, 0)),
                  _ROW_SPEC],
        out_specs=_ROW_SPEC,
        out_shape=jax.ShapeDtypeStruct((_NP2, 128), jnp.float32),
    )(x, x, w, disp)


def _combine_mm(acc, hs, disp, bp, wbd):
    return pl.pallas_call(
        _combine_mm_body, grid=_GRID,
        in_specs=[_ACC_SPEC, _ROW_SPEC, _ROW_SPEC,
                  pl.BlockSpec((1, 128), lambda i: (0, 0)),
                  pl.BlockSpec((128, 128), lambda i: (0, 0))],
        out_specs=_ROW_SPEC,
        out_shape=jax.ShapeDtypeStruct((_NP2, 128), jnp.float32),
    )(acc, hs, disp, bp, wbd)


def _final(acc, hs, disp, bp, blo, bhi, wfc, bfc):
    return pl.pallas_call(
        _final_body, grid=_GRID,
        in_specs=[_ACC_SPEC, _ROW_SPEC, _ROW_SPEC,
                  pl.BlockSpec((1, 128), lambda i: (0, 0)),
                  pl.BlockSpec((1, 1, _PBLK), lambda i: (i, 0, 0)),
                  pl.BlockSpec((1, 1, _PBLK), lambda i: (i, 0, 0)),
                  pl.BlockSpec((_H, 2), lambda i: (0, 0)),
                  pl.BlockSpec((1, 2), lambda i: (0, 0))],
        out_specs=pl.BlockSpec((_G, 2), lambda i: (0, 0)),
        out_shape=jax.ShapeDtypeStruct((_G, 2), jnp.float32),
        scratch_shapes=[pltpu.VMEM((_G, _H), jnp.float32),
                        pltpu.VMEM((1, _G), jnp.float32)],
    )(acc, hs, disp, bp, blo, bhi, wfc, bfc)


def _blockdiag(w):
    z = jnp.zeros((_H, _H), jnp.float32)
    return jnp.concatenate(
        [jnp.concatenate([w, z], axis=1),
         jnp.concatenate([z, w], axis=1)], axis=0)


def kernel(x, edge_index, batch, W1, b1, W2, b2, W3, b3, Wfc, bfc):
    src = edge_index[0].astype(jnp.int32)
    dst = edge_index[1].astype(jnp.int32)

    def sc_row(v):
        return jnp.where(v < _NP2, 2 * v, 2 * (v - _NP2) + 1)

    srcg = sc_row(src).reshape(_NC * _NS, _KCH, _CH)
    dstg = sc_row(dst).reshape(_NC * _NS, _KCH, _CH)
    dstl = dst.reshape(_NC * _NS, _KCH, _CH)

    def scat(hs_packed):
        acc = _edge_scatter_kernel()(hs_packed.reshape(_NPAD, _H), srcg, dstg)
        return acc.reshape(_NC, _NP2, 128)

    degp = _deg_hist_kernel()(dstl)
    disp = _dis_prep(degp)
    hs1 = _mm_scale(x, W1, disp)
    acc1 = scat(hs1)
    hs2 = _combine_mm(acc1, hs1, disp, jnp.tile(b1.reshape(1, _H), (1, 2)),
                      _blockdiag(W2))
    acc2 = scat(hs2)
    hs3 = _combine_mm(acc2, hs2, disp, jnp.tile(b2.reshape(1, _H), (1, 2)),
                      _blockdiag(W3))
    acc3 = scat(hs3)
    bpad = jnp.concatenate([batch.astype(jnp.int32),
                            jnp.full((_NPAD - _N,), -1, jnp.int32)])
    blo = bpad[:_NP2].reshape(_NGRID, 1, _PBLK)
    bhi = bpad[_NP2:].reshape(_NGRID, 1, _PBLK)
    return _final(acc3, hs3, disp, jnp.tile(b3.reshape(1, _H), (1, 2)),
                  blo, bhi, Wfc, bfc.reshape(1, 2))


# speedup vs baseline: 37.4006x; 1.0232x over previous
import functools

import jax
import jax.numpy as jnp
from jax import lax
from jax.experimental import pallas as pl
from jax.experimental.pallas import tpu as pltpu
from jax.experimental.pallas import tpu_sc as plsc

_N = 10000
_E = 320000
_DIN = 128
_H = 64
_G = 16

_NC, _NS = 2, 16
_CHR = 125
_CH = 128
_KCH = 80
_SINK = 10111
_NPAD = 10112
_STRIPE = _NPAD // _NS
_NP2 = _NPAD // 2
_NGRID = 8
_PBLK = _NP2 // _NGRID


def _sc_mesh():
    return plsc.VectorSubcoreMesh(core_axis_name="c", subcore_axis_name="s",
                                  num_cores=_NC, num_subcores=_NS)


@functools.cache
def _deg_hist_kernel():
    return pl.kernel(
        _deg_hist_body,
        out_type=jax.ShapeDtypeStruct((_NC, _NPAD, 16), jnp.float32),
        mesh=_sc_mesh(),
        scratch_types=[
            pltpu.VMEM((_KCH, _CH), jnp.int32),
            pltpu.VMEM((_CH, 16), jnp.float32),
            pltpu.VMEM((_CH, 16), jnp.float32),
            pltpu.VMEM_SHARED((_NPAD, 16), jnp.float32),
        ],
        compiler_params=pltpu.CompilerParams(use_tc_tiling_on_sc=False),
    )


def _deg_hist_body(dstg, out, dstv, onev, zb, acc):
    c = lax.axis_index("c")
    s = lax.axis_index("s")
    wid = c * _NS + s

    def _fill(r, carry):
        zb[r, pl.ds(0, 16)] = jnp.zeros((16,), jnp.float32)
        onev[r, pl.ds(0, 16)] = jnp.ones((16,), jnp.float32)
        return carry

    lax.fori_loop(0, _CH, _fill, 0)
    row0 = s * _STRIPE
    for j in range(_STRIPE // _CH):
        pltpu.sync_copy(zb, acc.at[pl.ds(row0 + j * _CH, _CH)])
    rem = _STRIPE % _CH
    if rem:
        pltpu.sync_copy(zb.at[pl.ds(0, rem)],
                        acc.at[pl.ds(row0 + (_STRIPE // _CH) * _CH, rem)])
    pltpu.sync_copy(dstg.at[wid], dstv)
    plsc.subcore_barrier()

    def _step(k, carry):
        pltpu.sync_copy(onev, acc.at[dstv.at[k]], add=True)
        return carry

    lax.fori_loop(0, _KCH, _step, 0)
    plsc.subcore_barrier()
    pltpu.sync_copy(acc.at[pl.ds(row0, _STRIPE)], out.at[c, pl.ds(row0, _STRIPE)])


@functools.cache
def _edge_scatter_kernel():
    return pl.kernel(
        _edge_scatter_body,
        out_type=jax.ShapeDtypeStruct((_NC, _NPAD, _H), jnp.float32),
        mesh=_sc_mesh(),
        scratch_types=[
            pltpu.VMEM((_KCH, _CH), jnp.int32),
            pltpu.VMEM((_KCH, _CH), jnp.int32),
            pltpu.VMEM((_CH, _H), jnp.float32),
            pltpu.VMEM((_CH, _H), jnp.float32),
            pltpu.VMEM((_CH, _H), jnp.float32),
            pltpu.VMEM_SHARED((_NPAD, _H), jnp.float32),
            pltpu.VMEM_SHARED((_NPAD, _H), jnp.float32),
            pltpu.SemaphoreType.DMA,
            pltpu.SemaphoreType.DMA,
        ],
        compiler_params=pltpu.CompilerParams(use_tc_tiling_on_sc=False),
    )


def _edge_scatter_body(hs, srcg, dstg, out, srcv, dstv, b0, b1, zb, acc, hs_sh,
                       semA, semB):
    c = lax.axis_index("c")
    s = lax.axis_index("s")
    wid = c * _NS + s
    row0 = s * _STRIPE

    d1 = pltpu.async_copy(srcg.at[wid], srcv, semA)
    d2 = pltpu.async_copy(dstg.at[wid], dstv, semA)
    d3 = pltpu.async_copy(hs.at[pl.ds(row0, _STRIPE)],
                          hs_sh.at[pl.ds(row0, _STRIPE)], semB)

    def _zrow(r, carry):
        for q in range(_H // 16):
            zb[r, pl.ds(q * 16, 16)] = jnp.zeros((16,), jnp.float32)
        return carry

    lax.fori_loop(0, _CH, _zrow, 0)
    for j in range(_STRIPE // _CH):
        pltpu.sync_copy(zb, acc.at[pl.ds(row0 + j * _CH, _CH)])
    rem = _STRIPE % _CH
    if rem:
        pltpu.sync_copy(zb.at[pl.ds(0, rem)],
                        acc.at[pl.ds(row0 + (_STRIPE // _CH) * _CH, rem)])
    d1.wait()
    d2.wait()
    d3.wait()
    plsc.subcore_barrier()

    pltpu.async_copy(hs_sh.at[srcv.at[0]], b0, semA)

    def _step(i, carry):
        k = i * 2
        pltpu.make_async_copy(hs_sh.at[srcv.at[k]], b0, semA).wait()
        pltpu.async_copy(hs_sh.at[srcv.at[k + 1]], b1, semB)
        pltpu.sync_copy(b0, acc.at[dstv.at[k]], add=True)
        pltpu.make_async_copy(hs_sh.at[srcv.at[k + 1]], b1, semB).wait()

        @pl.when(k + 2 < _KCH)
        def _():
            pltpu.async_copy(hs_sh.at[srcv.at[k + 2]], b0, semA)

        pltpu.sync_copy(b1, acc.at[dstv.at[k + 1]], add=True)
        return carry

    lax.fori_loop(0, _KCH // 2, _step, 0)
    plsc.subcore_barrier()
    pltpu.sync_copy(acc.at[pl.ds(row0, _STRIPE)], out.at[c, pl.ds(row0, _STRIPE)])


def _dis_prep_body(deg_ref, o_ref):
    deg = 1.0 + deg_ref[0, :, 0] + deg_ref[1, :, 0]
    dis = lax.rsqrt(deg)
    lo = jnp.broadcast_to(dis[:_NP2, None], (_NP2, _H))
    hi = jnp.broadcast_to(dis[_NP2:, None], (_NP2, _H))
    o_ref[...] = jnp.concatenate([lo, hi], axis=1)


def _mm_scale_body(xlo_ref, xhi_ref, w_ref, disp_ref, o_ref):
    i = pl.program_id(0)
    w = w_ref[...]
    dis_p = disp_ref[...]
    hlo = jnp.dot(xlo_ref[...], w, preferred_element_type=jnp.float32)
    hhi = jnp.dot(xhi_ref[...], w, preferred_element_type=jnp.float32)
    node_hi = _NP2 + i * _PBLK + lax.broadcasted_iota(jnp.int32, (_PBLK, _H), 0)
    hhi = jnp.where(node_hi < _N, hhi, 0.0)
    o_ref[...] = jnp.concatenate([hlo, hhi], axis=1) * dis_p


def _combine_mm_body(acc_ref, hs_ref, disp_ref, b_ref, w_ref, o_ref):
    dis_p = disp_ref[...]
    ssum = acc_ref[0] + acc_ref[1] + hs_ref[...]
    h = jnp.maximum(ssum * dis_p + b_ref[...], 0.0)
    o_ref[...] = jnp.dot(h, w_ref[...], preferred_element_type=jnp.float32) * dis_p


def _final_body(acc_ref, hs_ref, disp_ref, b_ref, blo_ref, bhi_ref, wfc_ref,
                bfc_ref, o_ref, sums_ref, cnt_ref):
    i = pl.program_id(0)
    dis_p = disp_ref[...]
    ssum = acc_ref[0] + acc_ref[1] + hs_ref[...]
    h3 = jnp.maximum(ssum * dis_p + b_ref[...], 0.0)
    gi = lax.broadcasted_iota(jnp.int32, (_G, _PBLK), 0)
    ohe = (gi == blo_ref[0, 0, :][None, :]).astype(jnp.float32)
    oho = (gi == bhi_ref[0, 0, :][None, :]).astype(jnp.float32)
    part = (jnp.dot(ohe, h3[:, :_H], preferred_element_type=jnp.float32)
            + jnp.dot(oho, h3[:, _H:], preferred_element_type=jnp.float32))
    pcnt = jnp.sum(ohe, axis=1) + jnp.sum(oho, axis=1)

    @pl.when(i == 0)
    def _():
        sums_ref[...] = part
        cnt_ref[...] = pcnt[None, :]

    @pl.when(i > 0)
    def _():
        sums_ref[...] += part
        cnt_ref[...] += pcnt[None, :]

    @pl.when(i == _NGRID - 1)
    def _():
        cnt = jnp.maximum(cnt_ref[0, :], 1.0)
        mean = sums_ref[...] / cnt[:, None]
        o_ref[...] = jnp.dot(mean, wfc_ref[...],
                             preferred_element_type=jnp.float32) + bfc_ref[...]


_GRID = (_NGRID,)
_ACC_SPEC = pl.BlockSpec((2, _PBLK, 128), lambda i: (0, i, 0))
_ROW_SPEC = pl.BlockSpec((_PBLK, 128), lambda i: (i, 0))


def _dis_prep(degp):
    return pl.pallas_call(
        _dis_prep_body, grid=(1,),
        in_specs=[pl.BlockSpec((2, _NPAD, 16), lambda i: (0, 0, 0))],
        out_specs=pl.BlockSpec((_NP2, 128), lambda i: (0, 0)),
        out_shape=jax.ShapeDtypeStruct((_NP2, 128), jnp.float32),
    )(degp)


def _mm_scale(x, w, disp):
    return pl.pallas_call(
        _mm_scale_body, grid=_GRID,
        in_specs=[pl.BlockSpec((_PBLK, _DIN), lambda i: (i, 0)),
                  pl.BlockSpec((_PBLK, _DIN), lambda i: (i + _NGRID, 0)),
                  pl.BlockSpec((_DIN, _H), lambda i: (0, 0)),
                  _ROW_SPEC],
        out_specs=_ROW_SPEC,
        out_shape=jax.ShapeDtypeStruct((_NP2, 128), jnp.float32),
    )(x, x, w, disp)


def _combine_mm(acc, hs, disp, bp, wbd):
    return pl.pallas_call(
        _combine_mm_body, grid=_GRID,
        in_specs=[_ACC_SPEC, _ROW_SPEC, _ROW_SPEC,
                  pl.BlockSpec((1, 128), lambda i: (0, 0)),
                  pl.BlockSpec((128, 128), lambda i: (0, 0))],
        out_specs=_ROW_SPEC,
        out_shape=jax.ShapeDtypeStruct((_NP2, 128), jnp.float32),
    )(acc, hs, disp, bp, wbd)


def _final(acc, hs, disp, bp, blo, bhi, wfc, bfc):
    return pl.pallas_call(
        _final_body, grid=_GRID,
        in_specs=[_ACC_SPEC, _ROW_SPEC, _ROW_SPEC,
                  pl.BlockSpec((1, 128), lambda i: (0, 0)),
                  pl.BlockSpec((1, 1, _PBLK), lambda i: (i, 0, 0)),
                  pl.BlockSpec((1, 1, _PBLK), lambda i: (i, 0, 0)),
                  pl.BlockSpec((_H, 2), lambda i: (0, 0)),
                  pl.BlockSpec((1, 2), lambda i: (0, 0))],
        out_specs=pl.BlockSpec((_G, 2), lambda i: (0, 0)),
        out_shape=jax.ShapeDtypeStruct((_G, 2), jnp.float32),
        scratch_shapes=[pltpu.VMEM((_G, _H), jnp.float32),
                        pltpu.VMEM((1, _G), jnp.float32)],
    )(acc, hs, disp, bp, blo, bhi, wfc, bfc)


def _blockdiag(w):
    z = jnp.zeros((_H, _H), jnp.float32)
    return jnp.concatenate(
        [jnp.concatenate([w, z], axis=1),
         jnp.concatenate([z, w], axis=1)], axis=0)


def kernel(x, edge_index, batch, W1, b1, W2, b2, W3, b3, Wfc, bfc):
    src = edge_index[0].astype(jnp.int32)
    dst = edge_index[1].astype(jnp.int32)

    def sc_row(v):
        return jnp.where(v < _NP2, 2 * v, 2 * (v - _NP2) + 1)

    def chunked(v, fill):
        return jnp.pad(v.reshape(_NC * _NS, _KCH, _CHR),
                       ((0, 0), (0, 0), (0, _CH - _CHR)), constant_values=fill)

    srcg = chunked(sc_row(src), 0)
    dstg = chunked(sc_row(dst), _SINK)
    dstl = chunked(dst, _SINK)

    def scat(hs_packed):
        acc = _edge_scatter_kernel()(hs_packed.reshape(_NPAD, _H), srcg, dstg)
        return acc.reshape(_NC, _NP2, 128)

    degp = _deg_hist_kernel()(dstl)
    disp = _dis_prep(degp)
    hs1 = _mm_scale(x, W1, disp)
    acc1 = scat(hs1)
    hs2 = _combine_mm(acc1, hs1, disp, jnp.tile(b1.reshape(1, _H), (1, 2)),
                      _blockdiag(W2))
    acc2 = scat(hs2)
    hs3 = _combine_mm(acc2, hs2, disp, jnp.tile(b2.reshape(1, _H), (1, 2)),
                      _blockdiag(W3))
    acc3 = scat(hs3)
    bpad = jnp.concatenate([batch.astype(jnp.int32),
                            jnp.full((_NPAD - _N,), -1, jnp.int32)])
    blo = bpad[:_NP2].reshape(_NGRID, 1, _PBLK)
    bhi = bpad[_NP2:].reshape(_NGRID, 1, _PBLK)
    return _final(acc3, hs3, disp, jnp.tile(b3.reshape(1, _H), (1, 2)),
                  blo, bhi, Wfc, bfc.reshape(1, 2))


# speedup vs baseline: 38.3457x; 1.0253x over previous
import functools

import jax
import jax.numpy as jnp
from jax import lax
from jax.experimental import pallas as pl
from jax.experimental.pallas import tpu as pltpu
from jax.experimental.pallas import tpu_sc as plsc

_N = 10000
_E = 320000
_DIN = 128
_H = 64
_G = 16

_NC, _NS = 2, 16
_CHR = 125
_CH = 128
_KCH = 80
_SINK = 10111
_NPAD = 10112
_STRIPE = _NPAD // _NS
_NP2 = _NPAD // 2
_NGRID = 8
_PBLK = _NP2 // _NGRID


def _sc_mesh():
    return plsc.VectorSubcoreMesh(core_axis_name="c", subcore_axis_name="s",
                                  num_cores=_NC, num_subcores=_NS)


@functools.cache
def _deg_hist_kernel():
    return pl.kernel(
        _deg_hist_body,
        out_type=jax.ShapeDtypeStruct((_NC, _NPAD, _H), jnp.float32),
        mesh=_sc_mesh(),
        scratch_types=[
            pltpu.VMEM((_KCH, _CH), jnp.int32),
            pltpu.VMEM((_CH, 16), jnp.float32),
            pltpu.VMEM((_CH, 16), jnp.float32),
            pltpu.VMEM_SHARED((_NPAD, 16), jnp.float32),
            pltpu.VMEM((_STRIPE, 16), jnp.float32),
            pltpu.VMEM((_STRIPE, _H), jnp.float32),
        ],
        compiler_params=pltpu.CompilerParams(use_tc_tiling_on_sc=False),
    )


def _deg_hist_body(dstg, out, dstv, onev, zb, acc, ebuf, obuf):
    c = lax.axis_index("c")
    s = lax.axis_index("s")
    wid = c * _NS + s

    def _fill(r, carry):
        zb[r, pl.ds(0, 16)] = jnp.zeros((16,), jnp.float32)
        onev[r, pl.ds(0, 16)] = jnp.ones((16,), jnp.float32)
        return carry

    lax.fori_loop(0, _CH, _fill, 0)
    row0 = s * _STRIPE
    for j in range(_STRIPE // _CH):
        pltpu.sync_copy(zb, acc.at[pl.ds(row0 + j * _CH, _CH)])
    rem = _STRIPE % _CH
    if rem:
        pltpu.sync_copy(zb.at[pl.ds(0, rem)],
                        acc.at[pl.ds(row0 + (_STRIPE // _CH) * _CH, rem)])
    pltpu.sync_copy(dstg.at[wid], dstv)
    plsc.subcore_barrier()

    def _step(k, carry):
        pltpu.sync_copy(onev, acc.at[dstv.at[k]], add=True)
        return carry

    lax.fori_loop(0, _KCH, _step, 0)
    plsc.subcore_barrier()
    pltpu.sync_copy(acc.at[pl.ds(row0, _STRIPE)], ebuf)

    def _rep(r, carry):
        v = ebuf[r, pl.ds(0, 16)]
        for q in range(_H // 16):
            obuf[r, pl.ds(q * 16, 16)] = v
        return carry

    lax.fori_loop(0, _STRIPE, _rep, 0)
    pltpu.sync_copy(obuf, out.at[c, pl.ds(row0, _STRIPE)])


@functools.cache
def _edge_scatter_kernel():
    return pl.kernel(
        _edge_scatter_body,
        out_type=jax.ShapeDtypeStruct((_NC, _NPAD, _H), jnp.float32),
        mesh=_sc_mesh(),
        scratch_types=[
            pltpu.VMEM((_KCH, _CH), jnp.int32),
            pltpu.VMEM((_KCH, _CH), jnp.int32),
            pltpu.VMEM((_CH, _H), jnp.float32),
            pltpu.VMEM((_CH, _H), jnp.float32),
            pltpu.VMEM((_CH, _H), jnp.float32),
            pltpu.VMEM_SHARED((_NPAD, _H), jnp.float32),
            pltpu.VMEM_SHARED((_NPAD, _H), jnp.float32),
            pltpu.SemaphoreType.DMA,
            pltpu.SemaphoreType.DMA,
        ],
        compiler_params=pltpu.CompilerParams(use_tc_tiling_on_sc=False),
    )


def _edge_scatter_body(hs, srcg, dstg, out, srcv, dstv, b0, b1, zb, acc, hs_sh,
                       semA, semB):
    c = lax.axis_index("c")
    s = lax.axis_index("s")
    wid = c * _NS + s
    row0 = s * _STRIPE

    d1 = pltpu.async_copy(srcg.at[wid], srcv, semA)
    d2 = pltpu.async_copy(dstg.at[wid], dstv, semA)
    d3 = pltpu.async_copy(hs.at[pl.ds(row0, _STRIPE)],
                          hs_sh.at[pl.ds(row0, _STRIPE)], semB)

    def _zrow(r, carry):
        for q in range(_H // 16):
            zb[r, pl.ds(q * 16, 16)] = jnp.zeros((16,), jnp.float32)
        return carry

    lax.fori_loop(0, _CH, _zrow, 0)
    for j in range(_STRIPE // _CH):
        pltpu.sync_copy(zb, acc.at[pl.ds(row0 + j * _CH, _CH)])
    rem = _STRIPE % _CH
    if rem:
        pltpu.sync_copy(zb.at[pl.ds(0, rem)],
                        acc.at[pl.ds(row0 + (_STRIPE // _CH) * _CH, rem)])
    d1.wait()
    d2.wait()
    d3.wait()
    plsc.subcore_barrier()

    pltpu.async_copy(hs_sh.at[srcv.at[0]], b0, semA)

    def _step(i, carry):
        k = i * 2
        pltpu.make_async_copy(hs_sh.at[srcv.at[k]], b0, semA).wait()
        pltpu.async_copy(hs_sh.at[srcv.at[k + 1]], b1, semB)
        pltpu.sync_copy(b0, acc.at[dstv.at[k]], add=True)
        pltpu.make_async_copy(hs_sh.at[srcv.at[k + 1]], b1, semB).wait()

        @pl.when(k + 2 < _KCH)
        def _():
            pltpu.async_copy(hs_sh.at[srcv.at[k + 2]], b0, semA)

        pltpu.sync_copy(b1, acc.at[dstv.at[k + 1]], add=True)
        return carry

    lax.fori_loop(0, _KCH // 2, _step, 0)
    plsc.subcore_barrier()
    pltpu.sync_copy(acc.at[pl.ds(row0, _STRIPE)], out.at[c, pl.ds(row0, _STRIPE)])


def _disp_from(dp_ref):
    return lax.rsqrt(1.0 + dp_ref[0] + dp_ref[1])


def _mm_scale_body(xlo_ref, xhi_ref, w_ref, dp_ref, o_ref):
    i = pl.program_id(0)
    w = w_ref[...]
    dis_p = _disp_from(dp_ref)
    hlo = jnp.dot(xlo_ref[...], w, preferred_element_type=jnp.float32)
    hhi = jnp.dot(xhi_ref[...], w, preferred_element_type=jnp.float32)
    node_hi = _NP2 + i * _PBLK + lax.broadcasted_iota(jnp.int32, (_PBLK, _H), 0)
    hhi = jnp.where(node_hi < _N, hhi, 0.0)
    o_ref[...] = jnp.concatenate([hlo, hhi], axis=1) * dis_p


def _combine_mm_body(acc_ref, hs_ref, dp_ref, b_ref, w_ref, o_ref):
    dis_p = _disp_from(dp_ref)
    ssum = acc_ref[0] + acc_ref[1] + hs_ref[...]
    h = jnp.maximum(ssum * dis_p + b_ref[...], 0.0)
    o_ref[...] = jnp.dot(h, w_ref[...], preferred_element_type=jnp.float32) * dis_p


def _final_body(acc_ref, hs_ref, dp_ref, b_ref, blo_ref, bhi_ref, wfc_ref,
                bfc_ref, o_ref, sums_ref, cnt_ref):
    i = pl.program_id(0)
    dis_p = _disp_from(dp_ref)
    ssum = acc_ref[0] + acc_ref[1] + hs_ref[...]
    h3 = jnp.maximum(ssum * dis_p + b_ref[...], 0.0)
    gi = lax.broadcasted_iota(jnp.int32, (_G, _PBLK), 0)
    ohe = (gi == blo_ref[0, 0, :][None, :]).astype(jnp.float32)
    oho = (gi == bhi_ref[0, 0, :][None, :]).astype(jnp.float32)
    part = (jnp.dot(ohe, h3[:, :_H], preferred_element_type=jnp.float32)
            + jnp.dot(oho, h3[:, _H:], preferred_element_type=jnp.float32))
    pcnt = jnp.sum(ohe, axis=1) + jnp.sum(oho, axis=1)

    @pl.when(i == 0)
    def _():
        sums_ref[...] = part
        cnt_ref[...] = pcnt[None, :]

    @pl.when(i > 0)
    def _():
        sums_ref[...] += part
        cnt_ref[...] += pcnt[None, :]

    @pl.when(i == _NGRID - 1)
    def _():
        cnt = jnp.maximum(cnt_ref[0, :], 1.0)
        mean = sums_ref[...] / cnt[:, None]
        o_ref[...] = jnp.dot(mean, wfc_ref[...],
                             preferred_element_type=jnp.float32) + bfc_ref[...]


_GRID = (_NGRID,)
_ACC_SPEC = pl.BlockSpec((2, _PBLK, 128), lambda i: (0, i, 0))
_ROW_SPEC = pl.BlockSpec((_PBLK, 128), lambda i: (i, 0))


def _mm_scale(x, w, degp):
    return pl.pallas_call(
        _mm_scale_body, grid=_GRID,
        in_specs=[pl.BlockSpec((_PBLK, _DIN), lambda i: (i, 0)),
                  pl.BlockSpec((_PBLK, _DIN), lambda i: (i + _NGRID, 0)),
                  pl.BlockSpec((_DIN, _H), lambda i: (0, 0)),
                  _ACC_SPEC],
        out_specs=_ROW_SPEC,
        out_shape=jax.ShapeDtypeStruct((_NP2, 128), jnp.float32),
    )(x, x, w, degp)


def _combine_mm(acc, hs, degp, bp, wbd):
    return pl.pallas_call(
        _combine_mm_body, grid=_GRID,
        in_specs=[_ACC_SPEC, _ROW_SPEC, _ACC_SPEC,
                  pl.BlockSpec((1, 128), lambda i: (0, 0)),
                  pl.BlockSpec((128, 128), lambda i: (0, 0))],
        out_specs=_ROW_SPEC,
        out_shape=jax.ShapeDtypeStruct((_NP2, 128), jnp.float32),
    )(acc, hs, degp, bp, wbd)


def _final(acc, hs, degp, bp, blo, bhi, wfc, bfc):
    return pl.pallas_call(
        _final_body, grid=_GRID,
        in_specs=[_ACC_SPEC, _ROW_SPEC, _ACC_SPEC,
                  pl.BlockSpec((1, 128), lambda i: (0, 0)),
                  pl.BlockSpec((1, 1, _PBLK), lambda i: (i, 0, 0)),
                  pl.BlockSpec((1, 1, _PBLK), lambda i: (i, 0, 0)),
                  pl.BlockSpec((_H, 2), lambda i: (0, 0)),
                  pl.BlockSpec((1, 2), lambda i: (0, 0))],
        out_specs=pl.BlockSpec((_G, 2), lambda i: (0, 0)),
        out_shape=jax.ShapeDtypeStruct((_G, 2), jnp.float32),
        scratch_shapes=[pltpu.VMEM((_G, _H), jnp.float32),
                        pltpu.VMEM((1, _G), jnp.float32)],
    )(acc, hs, degp, bp, blo, bhi, wfc, bfc)


def _blockdiag(w):
    z = jnp.zeros((_H, _H), jnp.float32)
    return jnp.concatenate(
        [jnp.concatenate([w, z], axis=1),
         jnp.concatenate([z, w], axis=1)], axis=0)


def kernel(x, edge_index, batch, W1, b1, W2, b2, W3, b3, Wfc, bfc):
    src = edge_index[0].astype(jnp.int32)
    dst = edge_index[1].astype(jnp.int32)

    def sc_row(v):
        return jnp.where(v < _NP2, 2 * v, 2 * (v - _NP2) + 1)

    def chunked(v, fill):
        return jnp.pad(v.reshape(_NC * _NS, _KCH, _CHR),
                       ((0, 0), (0, 0), (0, _CH - _CHR)), constant_values=fill)

    srcg = chunked(sc_row(src), 0)
    dstg = chunked(sc_row(dst), _SINK)
    def scat(hs_packed):
        acc = _edge_scatter_kernel()(hs_packed.reshape(_NPAD, _H), srcg, dstg)
        return acc.reshape(_NC, _NP2, 128)

    degp = _deg_hist_kernel()(dstg).reshape(_NC, _NP2, 128)
    hs1 = _mm_scale(x, W1, degp)
    acc1 = scat(hs1)
    hs2 = _combine_mm(acc1, hs1, degp, jnp.tile(b1.reshape(1, _H), (1, 2)),
                      _blockdiag(W2))
    acc2 = scat(hs2)
    hs3 = _combine_mm(acc2, hs2, degp, jnp.tile(b2.reshape(1, _H), (1, 2)),
                      _blockdiag(W3))
    acc3 = scat(hs3)
    bpad = jnp.concatenate([batch.astype(jnp.int32),
                            jnp.full((_NPAD - _N,), -1, jnp.int32)])
    blo = bpad[:_NP2].reshape(_NGRID, 1, _PBLK)
    bhi = bpad[_NP2:].reshape(_NGRID, 1, _PBLK)
    return _final(acc3, hs3, degp, jnp.tile(b3.reshape(1, _H), (1, 2)),
                  blo, bhi, Wfc, bfc.reshape(1, 2))


# speedup vs baseline: 38.4037x; 1.0015x over previous
import functools

import jax
import jax.numpy as jnp
from jax import lax
from jax.experimental import pallas as pl
from jax.experimental.pallas import tpu as pltpu
from jax.experimental.pallas import tpu_sc as plsc

_N = 10000
_E = 320000
_DIN = 128
_H = 64
_G = 16

_NC, _NS = 2, 16
_CHR = 125
_CH = 128
_KCH = 80
_SINK = 10111
_NPAD = 10112
_STRIPE = _NPAD // _NS
_NP2 = _NPAD // 2
_NGRID = 8
_PBLK = _NP2 // _NGRID


def _sc_mesh():
    return plsc.VectorSubcoreMesh(core_axis_name="c", subcore_axis_name="s",
                                  num_cores=_NC, num_subcores=_NS)


@functools.cache
def _deg_hist_kernel():
    return pl.kernel(
        _deg_hist_body,
        out_type=jax.ShapeDtypeStruct((_NC, _NPAD, _H), jnp.float32),
        mesh=_sc_mesh(),
        scratch_types=[
            pltpu.VMEM((_KCH, _CH), jnp.int32),
            pltpu.VMEM((_CH, 16), jnp.float32),
            pltpu.VMEM((_CH, 16), jnp.float32),
            pltpu.VMEM_SHARED((_NPAD, 16), jnp.float32),
            pltpu.VMEM((_STRIPE, 16), jnp.float32),
            pltpu.VMEM((_STRIPE, _H), jnp.float32),
        ],
        compiler_params=pltpu.CompilerParams(use_tc_tiling_on_sc=False),
    )


def _deg_hist_body(dstg, out, dstv, onev, zb, acc, ebuf, obuf):
    c = lax.axis_index("c")
    s = lax.axis_index("s")
    wid = c * _NS + s

    def _fill(r, carry):
        zb[r, pl.ds(0, 16)] = jnp.zeros((16,), jnp.float32)
        onev[r, pl.ds(0, 16)] = jnp.ones((16,), jnp.float32)
        return carry

    lax.fori_loop(0, _CH, _fill, 0)
    row0 = s * _STRIPE
    for j in range(_STRIPE // _CH):
        pltpu.sync_copy(zb, acc.at[pl.ds(row0 + j * _CH, _CH)])
    rem = _STRIPE % _CH
    if rem:
        pltpu.sync_copy(zb.at[pl.ds(0, rem)],
                        acc.at[pl.ds(row0 + (_STRIPE // _CH) * _CH, rem)])
    pltpu.sync_copy(dstg.at[wid], dstv)
    plsc.subcore_barrier()

    def _step(k, carry):
        pltpu.sync_copy(onev, acc.at[dstv.at[k]], add=True)
        return carry

    lax.fori_loop(0, _KCH, _step, 0)
    plsc.subcore_barrier()
    pltpu.sync_copy(acc.at[pl.ds(row0, _STRIPE)], ebuf)

    def _rep(r, carry):
        v = ebuf[r, pl.ds(0, 16)]
        for q in range(_H // 16):
            obuf[r, pl.ds(q * 16, 16)] = v
        return carry

    lax.fori_loop(0, _STRIPE, _rep, 0)
    pltpu.sync_copy(obuf, out.at[c, pl.ds(row0, _STRIPE)])


@functools.cache
def _edge_scatter_kernel():
    return pl.kernel(
        _edge_scatter_body,
        out_type=jax.ShapeDtypeStruct((_NC, _NPAD, _H), jnp.float32),
        mesh=_sc_mesh(),
        scratch_types=[
            pltpu.VMEM((_KCH, _CH), jnp.int32),
            pltpu.VMEM((_KCH, _CH), jnp.int32),
            pltpu.VMEM((_CH, _H), jnp.float32),
            pltpu.VMEM((_CH, _H), jnp.float32),
            pltpu.VMEM((_CH, _H), jnp.float32),
            pltpu.VMEM_SHARED((_NPAD, _H), jnp.float32),
            pltpu.VMEM_SHARED((_NPAD, _H), jnp.float32),
            pltpu.SemaphoreType.DMA,
            pltpu.SemaphoreType.DMA,
        ],
        compiler_params=pltpu.CompilerParams(use_tc_tiling_on_sc=False),
    )


def _edge_scatter_body(hs, srcg, dstg, out, srcv, dstv, b0, b1, zb, acc, hs_sh,
                       semA, semB):
    c = lax.axis_index("c")
    s = lax.axis_index("s")
    wid = c * _NS + s
    row0 = s * _STRIPE

    d1 = pltpu.async_copy(srcg.at[wid], srcv, semA)
    d2 = pltpu.async_copy(dstg.at[wid], dstv, semA)
    d3 = pltpu.async_copy(hs.at[pl.ds(row0, _STRIPE)],
                          hs_sh.at[pl.ds(row0, _STRIPE)], semB)

    def _zrow(r, carry):
        for q in range(_H // 16):
            zb[r, pl.ds(q * 16, 16)] = jnp.zeros((16,), jnp.float32)
        return carry

    lax.fori_loop(0, _CH, _zrow, 0)
    for j in range(_STRIPE // _CH):
        pltpu.sync_copy(zb, acc.at[pl.ds(row0 + j * _CH, _CH)])
    rem = _STRIPE % _CH
    if rem:
        pltpu.sync_copy(zb.at[pl.ds(0, rem)],
                        acc.at[pl.ds(row0 + (_STRIPE // _CH) * _CH, rem)])
    d1.wait()
    d2.wait()
    d3.wait()
    plsc.subcore_barrier()

    pltpu.async_copy(hs_sh.at[srcv.at[0]], b0, semA)

    def _step(i, carry):
        k = i * 2
        pltpu.make_async_copy(hs_sh.at[srcv.at[k]], b0, semA).wait()
        pltpu.async_copy(hs_sh.at[srcv.at[k + 1]], b1, semB)
        pltpu.sync_copy(b0, acc.at[dstv.at[k]], add=True)
        pltpu.make_async_copy(hs_sh.at[srcv.at[k + 1]], b1, semB).wait()

        @pl.when(k + 2 < _KCH)
        def _():
            pltpu.async_copy(hs_sh.at[srcv.at[k + 2]], b0, semA)

        pltpu.sync_copy(b1, acc.at[dstv.at[k + 1]], add=True)
        return carry

    lax.fori_loop(0, _KCH // 2, _step, 0)
    plsc.subcore_barrier()
    pltpu.sync_copy(acc.at[pl.ds(row0, _STRIPE)], out.at[c, pl.ds(row0, _STRIPE)])


def _disp_from(dp_ref):
    return lax.rsqrt(1.0 + dp_ref[0] + dp_ref[1])


def _mm_scale_body(xlo_ref, xhi_ref, w_ref, dp_ref, o_ref):
    i = pl.program_id(0)
    w = w_ref[...]
    dis_p = _disp_from(dp_ref)
    hlo = jnp.dot(xlo_ref[...], w, preferred_element_type=jnp.float32)
    hhi = jnp.dot(xhi_ref[...], w, preferred_element_type=jnp.float32)
    node_hi = _NP2 + i * _PBLK + lax.broadcasted_iota(jnp.int32, (_PBLK, _H), 0)
    hhi = jnp.where(node_hi < _N, hhi, 0.0)
    o_ref[...] = jnp.concatenate([hlo, hhi], axis=1) * dis_p


def _combine_mm_body(acc_ref, hs_ref, dp_ref, b_ref, w_ref, o_ref):
    dis_p = _disp_from(dp_ref)
    ssum = acc_ref[0] + acc_ref[1] + hs_ref[...]
    h = jnp.maximum(ssum * dis_p + b_ref[...], 0.0)
    o_ref[...] = jnp.dot(h, w_ref[...], preferred_element_type=jnp.float32) * dis_p


def _final_body(acc_ref, hs_ref, dp_ref, b_ref, blo_ref, bhi_ref, wfc_ref,
                bfc_ref, o_ref, sums_ref, cnt_ref):
    i = pl.program_id(0)
    dis_p = _disp_from(dp_ref)
    ssum = acc_ref[0] + acc_ref[1] + hs_ref[...]
    h3 = jnp.maximum(ssum * dis_p + b_ref[...], 0.0)
    gi = lax.broadcasted_iota(jnp.int32, (_G, _PBLK), 0)
    ohe = (gi == blo_ref[0, 0, :][None, :]).astype(jnp.float32)
    oho = (gi == bhi_ref[0, 0, :][None, :]).astype(jnp.float32)
    part = (jnp.dot(ohe, h3[:, :_H], preferred_element_type=jnp.float32)
            + jnp.dot(oho, h3[:, _H:], preferred_element_type=jnp.float32))
    pcnt = jnp.sum(ohe, axis=1) + jnp.sum(oho, axis=1)

    @pl.when(i == 0)
    def _():
        sums_ref[...] = part
        cnt_ref[...] = pcnt[None, :]

    @pl.when(i > 0)
    def _():
        sums_ref[...] += part
        cnt_ref[...] += pcnt[None, :]

    @pl.when(i == _NGRID - 1)
    def _():
        cnt = jnp.maximum(cnt_ref[0, :], 1.0)
        mean = sums_ref[...] / cnt[:, None]
        o_ref[...] = jnp.dot(mean, wfc_ref[...],
                             preferred_element_type=jnp.float32) + bfc_ref[...]


_GRID = (_NGRID,)
_ACC_SPEC = pl.BlockSpec((2, _PBLK, 128), lambda i: (0, i, 0))
_ROW_SPEC = pl.BlockSpec((_PBLK, 128), lambda i: (i, 0))


def _mm_scale(x, w, degp):
    return pl.pallas_call(
        _mm_scale_body, grid=_GRID,
        in_specs=[pl.BlockSpec((_PBLK, _DIN), lambda i: (i, 0)),
                  pl.BlockSpec((_PBLK, _DIN), lambda i: (i + _NGRID, 0)),
                  pl.BlockSpec((_DIN, _H), lambda i: (0, 0)),
                  _ACC_SPEC],
        out_specs=_ROW_SPEC,
        out_shape=jax.ShapeDtypeStruct((_NP2, 128), jnp.float32),
    )(x, x, w, degp)


def _combine_mm(acc, hs, degp, bp, wbd):
    return pl.pallas_call(
        _combine_mm_body, grid=_GRID,
        in_specs=[_ACC_SPEC, _ROW_SPEC, _ACC_SPEC,
                  pl.BlockSpec((1, 128), lambda i: (0, 0)),
                  pl.BlockSpec((128, 128), lambda i: (0, 0))],
        out_specs=_ROW_SPEC,
        out_shape=jax.ShapeDtypeStruct((_NP2, 128), jnp.float32),
    )(acc, hs, degp, bp, wbd)


def _final(acc, hs, degp, bp, blo, bhi, wfc, bfc):
    return pl.pallas_call(
        _final_body, grid=_GRID,
        in_specs=[_ACC_SPEC, _ROW_SPEC, _ACC_SPEC,
                  pl.BlockSpec((1, 128), lambda i: (0, 0)),
                  pl.BlockSpec((1, 1, _PBLK), lambda i: (i, 0, 0)),
                  pl.BlockSpec((1, 1, _PBLK), lambda i: (i, 0, 0)),
                  pl.BlockSpec((_H, 2), lambda i: (0, 0)),
                  pl.BlockSpec((1, 2), lambda i: (0, 0))],
        out_specs=pl.BlockSpec((_G, 2), lambda i: (0, 0)),
        out_shape=jax.ShapeDtypeStruct((_G, 2), jnp.float32),
        scratch_shapes=[pltpu.VMEM((_G, _H), jnp.float32),
                        pltpu.VMEM((1, _G), jnp.float32)],
    )(acc, hs, degp, bp, blo, bhi, wfc, bfc)


def _blockdiag(w):
    z = jnp.zeros((_H, _H), jnp.float32)
    return jnp.concatenate(
        [jnp.concatenate([w, z], axis=1),
         jnp.concatenate([z, w], axis=1)], axis=0)


def kernel(x, edge_index, batch, W1, b1, W2, b2, W3, b3, Wfc, bfc):
    src = edge_index[0].astype(jnp.int32)
    dst = edge_index[1].astype(jnp.int32)

    def sc_row(v):
        return jnp.where(v < _NP2, 2 * v, 2 * (v - _NP2) + 1)

    def chunked(v, fill):
        return jnp.pad(v.reshape(_NC * _NS, _KCH, _CHR),
                       ((0, 0), (0, 0), (0, _CH - _CHR)), constant_values=fill)

    srcg = sc_row(chunked(src, 0))
    dstg = sc_row(chunked(dst, _SINK))
    def scat(hs_packed):
        acc = _edge_scatter_kernel()(hs_packed.reshape(_NPAD, _H), srcg, dstg)
        return acc.reshape(_NC, _NP2, 128)

    degp = _deg_hist_kernel()(dstg).reshape(_NC, _NP2, 128)
    hs1 = _mm_scale(x, W1, degp)
    acc1 = scat(hs1)
    hs2 = _combine_mm(acc1, hs1, degp, jnp.tile(b1.reshape(1, _H), (1, 2)),
                      _blockdiag(W2))
    acc2 = scat(hs2)
    hs3 = _combine_mm(acc2, hs2, degp, jnp.tile(b2.reshape(1, _H), (1, 2)),
                      _blockdiag(W3))
    acc3 = scat(hs3)
    bpad = jnp.concatenate([batch.astype(jnp.int32),
                            jnp.full((_NPAD - _N,), -1, jnp.int32)])
    blo = bpad[:_NP2].reshape(_NGRID, 1, _PBLK)
    bhi = bpad[_NP2:].reshape(_NGRID, 1, _PBLK)
    return _final(acc3, hs3, degp, jnp.tile(b3.reshape(1, _H), (1, 2)),
                  blo, bhi, Wfc, bfc.reshape(1, 2))
